# baseline (speedup 1.0000x reference)
import functools

import jax
import jax.numpy as jnp
from jax import lax
from jax.experimental import pallas as pl
from jax.experimental.pallas import tpu as pltpu, tpu_sc as plsc

NC = 2
NS = 16
CW = 80
NSPLIT = 5000
NACC = 5128
NDRAIN = 5120
QR = 64


def _wxh_body(x_ref, w_ref, b_ref, o_ref):
    o_ref[...] = (
        jnp.dot(x_ref[...], w_ref[...], preferred_element_type=jnp.float32)
        + b_ref[...]
    )


def _filter_body(rbf_ref, w1_ref, b1_ref, w2_ref, b2_ref, o_ref):
    h = (
        jnp.dot(rbf_ref[...], w1_ref[...], preferred_element_type=jnp.float32)
        + b1_ref[...]
    )
    h = h * jax.nn.sigmoid(h)
    o_ref[...] = (
        jnp.dot(h, w2_ref[...], preferred_element_type=jnp.float32) + b2_ref[...]
    )


def _ln_body(x_ref, a_ref, a2_ref, g_ref, b_ref, o_ref):
    y = x_ref[...] + a_ref[0] + a2_ref[0]
    mu = jnp.mean(y, axis=1, keepdims=True)
    yc = y - mu
    var = jnp.mean(yc * yc, axis=1, keepdims=True)
    o_ref[...] = yc * lax.rsqrt(var + 1e-5) * g_ref[...] + b_ref[...]


def _sc_body(n_chunks, koff, f_h, wxh_h, ii_h, j_h, out_h,
             ii3, jidx, f_v, rows_v, buf_v, acc, gsem, fsem, ssem, iisem):
    ci = lax.axis_index("c")
    si = lax.axis_index("s")
    drain_per_tile = NDRAIN // NS
    n_q = drain_per_tile // QR

    def zrow(r, c):
        for cb in range(8):
            buf_v[r, pl.ds(cb * 16, 16)] = jnp.zeros((16,), jnp.float32)
        return c

    lax.fori_loop(0, QR, zrow, 0)
    for q in range(n_q):
        pltpu.sync_copy(buf_v, acc.at[pl.ds(si * drain_per_tile + q * QR, QR)])
    plsc.subcore_barrier()

    pltpu.sync_copy(j_h.at[si], jidx)

    def issue_gather(k, rv):
        return pltpu.async_copy(
            wxh_h.at[jidx.at[pl.ds((koff + k) * CW, CW)]], rv, gsem)

    def issue_f(k, fv):
        return pltpu.async_copy(f_h.at[si, k], fv, fsem)

    def issue_ii(k):
        return pltpu.async_copy(ii_h.at[ci, si, koff + k],
                                ii3.at[lax.rem(k, 3)], iisem)

    def wait(sem, dst):
        pltpu.make_async_copy(wxh_h.at[pl.ds(0, dst.shape[0])], dst, sem).wait()

    issue_ii(0)
    issue_ii(1)
    issue_gather(0, rows_v.at[0])
    issue_f(0, f_v.at[0])

    def halfstep(k, p, q):
        @pl.when(k >= 1)
        def _():
            pltpu.make_async_copy(rows_v.at[q], acc.at[pl.ds(0, CW)],
                                  ssem).wait()

        @pl.when(k + 1 < n_chunks)
        def _():
            issue_gather(k + 1, rows_v.at[q])
            issue_f(k + 1, f_v.at[q])

        @pl.when(k + 2 < n_chunks)
        def _():
            issue_ii(k + 2)

        wait(gsem, rows_v.at[p])
        wait(fsem, f_v.at[p])

        def mrow(r, c2):
            for cb in range(8):
                sl = pl.ds(cb * 16, 16)
                rows_v[p, r, sl] = rows_v[p, r, sl] * f_v[p, r, sl]
            return c2

        lax.fori_loop(0, CW, mrow, 0)
        pltpu.make_async_copy(ii_h.at[ci, si, koff + k], ii3.at[lax.rem(k, 3)],
                              iisem).wait()
        pltpu.async_copy(rows_v.at[p], acc.at[ii3.at[lax.rem(k, 3)]], ssem,
                         add=True)

    def step(g, c):
        halfstep(2 * g, 0, 1)
        halfstep(2 * g + 1, 1, 0)
        return c

    lax.fori_loop(0, n_chunks // 2, step, 0)
    if n_chunks % 2:
        halfstep(n_chunks - 1, 0, 1)
    pltpu.make_async_copy(rows_v.at[(n_chunks - 1) % 2], acc.at[pl.ds(0, CW)],
                          ssem).wait()
    plsc.subcore_barrier()

    for q in range(n_q):
        sl = pl.ds(si * drain_per_tile + q * QR, QR)
        pltpu.sync_copy(acc.at[sl], buf_v)
        pltpu.sync_copy(buf_v, out_h.at[ci, sl])


def kernel(x, i, j, rbf, W1, b1, W2, b2, Wl, bl, gamma, beta):
    n, d = x.shape
    e = i.shape[0]
    k_rbf = rbf.shape[1]
    n_chunks = e // (NS * CW)

    i32 = i.astype(jnp.int32)
    i_lo = jnp.where(i32 < NSPLIT, i32, NSPLIT)
    i_hi = jnp.where(i32 >= NSPLIT, i32 - NSPLIT, NSPLIT)
    nslab = 2
    ii4 = jnp.stack([i_lo, i_hi]).reshape(NC, NS, n_chunks, CW)
    j2 = j.astype(jnp.int32).reshape(NS, n_chunks * CW)

    nb = 2000
    wxh = pl.pallas_call(
        _wxh_body,
        grid=(n // nb,),
        in_specs=[
            pl.BlockSpec((nb, d), lambda g: (g, 0)),
            pl.BlockSpec((d, d), lambda g: (0, 0)),
            pl.BlockSpec((1, d), lambda g: (0, 0)),
        ],
        out_specs=pl.BlockSpec((nb, d), lambda g: (g, 0)),
        out_shape=jax.ShapeDtypeStruct((n, d), jnp.float32),
    )(x, Wl, bl.reshape(1, d))

    es = e // nslab
    ns_chunks = es // (NS * CW)
    eb = 2000
    bpt = (es // NS) // eb
    mesh = plsc.VectorSubcoreMesh(core_axis_name="c", subcore_axis_name="s")

    aggs = []
    for s in range(nslab):
        f_s = pl.pallas_call(
            _filter_body,
            grid=(es // eb,),
            in_specs=[
                pl.BlockSpec(
                    (eb, k_rbf),
                    lambda g, s=s: ((g // bpt) * (nslab * bpt) + s * bpt
                                    + g % bpt, 0)),
                pl.BlockSpec((k_rbf, d), lambda g: (0, 0)),
                pl.BlockSpec((1, d), lambda g: (0, 0)),
                pl.BlockSpec((d, d), lambda g: (0, 0)),
                pl.BlockSpec((1, d), lambda g: (0, 0)),
            ],
            out_specs=pl.BlockSpec((eb, d), lambda g: (g, 0)),
            out_shape=jax.ShapeDtypeStruct((es, d), jnp.float32),
        )(lax.slice_in_dim(rbf, s * es, (s + 1) * es),
          W1, b1.reshape(1, d), W2, b2.reshape(1, d))

        f4_s = f_s.reshape(NS, ns_chunks, CW, d)

        agg_s = pl.kernel(
            functools.partial(_sc_body, ns_chunks, s * ns_chunks),
            out_type=jax.ShapeDtypeStruct((NC, NDRAIN, d), jnp.float32),
            mesh=mesh,
            scratch_types=[
                pltpu.VMEM((3, CW), jnp.int32),
                pltpu.VMEM((n_chunks * CW,), jnp.int32),
                pltpu.VMEM((2, CW, d), jnp.float32),
                pltpu.VMEM((2, CW, d), jnp.float32),
                pltpu.VMEM((QR, d), jnp.float32),
                pltpu.VMEM_SHARED((NACC, d), jnp.float32),
                pltpu.SemaphoreType.DMA,
                pltpu.SemaphoreType.DMA,
                pltpu.SemaphoreType.DMA,
                pltpu.SemaphoreType.DMA,
            ],
        )(f4_s, wxh, ii4, j2)
        aggs.append(agg_s)

    nlb = 1000
    out = pl.pallas_call(
        _ln_body,
        grid=(n // nlb,),
        in_specs=[
            pl.BlockSpec((nlb, d), lambda g: (g, 0)),
            pl.BlockSpec((1, nlb, d), lambda g: (g // 5, g % 5, 0)),
            pl.BlockSpec((1, nlb, d), lambda g: (g // 5, g % 5, 0)),
            pl.BlockSpec((1, d), lambda g: (0, 0)),
            pl.BlockSpec((1, d), lambda g: (0, 0)),
        ],
        out_specs=pl.BlockSpec((nlb, d), lambda g: (g, 0)),
        out_shape=jax.ShapeDtypeStruct((n, d), jnp.float32),
    )(x, aggs[0], aggs[1], gamma.reshape(1, d), beta.reshape(1, d))
    return out


# speedup vs baseline: 1.2587x; 1.2587x over previous
import functools

import jax
import jax.numpy as jnp
from jax import lax
from jax.experimental import pallas as pl
from jax.experimental.pallas import tpu as pltpu, tpu_sc as plsc

NC = 2
NS = 16
CW = 80
NSPLIT = 5000
NACC = 5128
NDRAIN = 5120
QR = 16
EPT = 20000
SEG = 4000
NSEG = EPT // SEG
TRASH = 4090


def _wxh_body(x_ref, w_ref, b_ref, o_ref):
    o_ref[...] = (
        jnp.dot(x_ref[...], w_ref[...], preferred_element_type=jnp.float32)
        + b_ref[...]
    )


def _filter_body(rbf_ref, w1_ref, b1_ref, w2_ref, b2_ref, o_ref):
    h = (
        jnp.dot(rbf_ref[...], w1_ref[...], preferred_element_type=jnp.float32)
        + b1_ref[...]
    )
    h = h * jax.nn.sigmoid(h)
    o_ref[...] = (
        jnp.dot(h, w2_ref[...], preferred_element_type=jnp.float32) + b2_ref[...]
    )


def _ln_body(x_ref, a_ref, g_ref, b_ref, o_ref):
    y = x_ref[...] + a_ref[0]
    mu = jnp.mean(y, axis=1, keepdims=True)
    yc = y - mu
    var = jnp.mean(yc * yc, axis=1, keepdims=True)
    o_ref[...] = yc * lax.rsqrt(var + 1e-5) * g_ref[...] + b_ref[...]


def _sc_body(f_h, wxh_h, i_h, j_h, out_h,
             iraw, jidx, comp, fidx2, jrow2, irow3, f_v, rows_v, buf_v, acc,
             gsem, fsem, ssem):
    ci = lax.axis_index("c")
    si = lax.axis_index("s")
    ept = EPT
    drain_per_tile = NDRAIN // NS
    n_q = drain_per_tile // QR
    base = ci * NSPLIT

    def zrow(r, c):
        for cb in range(8):
            buf_v[r, pl.ds(cb * 16, 16)] = jnp.zeros((16,), jnp.float32)
        return c

    lax.fori_loop(0, QR, zrow, 0)
    for q in range(n_q):
        pltpu.sync_copy(buf_v, acc.at[pl.ds(si * drain_per_tile + q * QR, QR)])
    plsc.subcore_barrier()

    pltpu.sync_copy(i_h.at[si], iraw)
    pltpu.sync_copy(j_h.at[si], jidx)

    def build_idx(k, cnt, q, m3):
        for cb in range(CW // 16):
            sl = pl.ds(cb * 16, 16)
            ids = comp[pl.ds(k * CW + cb * 16, 16)]
            fidx2[q, sl] = ids + si * ept
            jrow2[q, sl] = plsc.load_gather(jidx, [ids])
            iv = plsc.load_gather(iraw, [ids])
            pos = k * CW + cb * 16 + lax.iota(jnp.int32, 16)
            irow3[m3, sl] = jnp.where(pos < cnt, iv - base, NSPLIT)

    def issue_f(q, fv):
        return pltpu.async_copy(f_h.at[fidx2.at[q]], fv, fsem)

    def issue_wxh(q, rv):
        return pltpu.async_copy(wxh_h.at[jrow2.at[q]], rv, gsem)

    def wait(sem, dst):
        pltpu.make_async_copy(wxh_h.at[pl.ds(0, dst.shape[0])], dst, sem).wait()

    iota16 = lax.iota(jnp.int32, 16)

    for sg in range(NSEG):
        def czero(g, c):
            comp[pl.ds(g * 16, 16)] = jnp.zeros((16,), jnp.int32)
            return c

        lax.fori_loop(0, (SEG + CW) // 16, czero, 0)

        def cgroup(g, off):
            v = iraw[pl.ds(sg * SEG + g * 16, 16)]
            m = (v >= base) & (v < base + NSPLIT)
            ids = sg * SEG + g * 16 + iota16
            pfx = plsc.cumsum(m.astype(jnp.int32))
            pos = jnp.where(m, off + pfx - 1, TRASH)
            plsc.store_scatter(comp, [pos], ids)
            return off + pfx[15]

        cnt = lax.fori_loop(0, SEG // 16, cgroup, 0)
        nck = (cnt + CW - 1) // CW

        @pl.when(nck > 0)
        def _():
            build_idx(0, cnt, 0, 0)
            issue_f(0, f_v.at[0])
            issue_wxh(0, rows_v.at[0])

        def halfstep(k, p, q):
            @pl.when(k >= 1)
            def _():
                pltpu.make_async_copy(rows_v.at[q], acc.at[pl.ds(0, CW)],
                                      ssem).wait()

            @pl.when(k + 1 < nck)
            def _():
                build_idx(k + 1, cnt, q, lax.rem(k + 1, 3))
                issue_f(q, f_v.at[q])
                issue_wxh(q, rows_v.at[q])

            wait(fsem, f_v.at[p])
            wait(gsem, rows_v.at[p])

            def mrow(r, c2):
                for cb in range(8):
                    sl = pl.ds(cb * 16, 16)
                    rows_v[p, r, sl] = rows_v[p, r, sl] * f_v[p, r, sl]
                return c2

            lax.fori_loop(0, CW, mrow, 0)
            pltpu.async_copy(rows_v.at[p], acc.at[irow3.at[lax.rem(k, 3)]],
                             ssem, add=True)

        def pair(g, c):
            halfstep(2 * g, 0, 1)

            @pl.when(2 * g + 1 < nck)
            def _():
                halfstep(2 * g + 1, 1, 0)

            return c

        lax.fori_loop(0, (nck + 1) // 2, pair, 0)

        @pl.when(nck > 0)
        def _():
            pltpu.make_async_copy(rows_v.at[lax.rem(nck - 1, 2)],
                                  acc.at[pl.ds(0, CW)], ssem).wait()

    plsc.subcore_barrier()

    for q in range(n_q):
        sl = pl.ds(si * drain_per_tile + q * QR, QR)
        pltpu.sync_copy(acc.at[sl], buf_v)
        pltpu.sync_copy(buf_v, out_h.at[ci, sl])


def kernel(x, i, j, rbf, W1, b1, W2, b2, Wl, bl, gamma, beta):
    n, d = x.shape
    e = i.shape[0]
    k_rbf = rbf.shape[1]
    n_chunks = e // (NS * CW)

    i2 = i.astype(jnp.int32).reshape(NS, EPT)
    j2 = j.astype(jnp.int32).reshape(NS, EPT)

    nb = 2000
    wxh = pl.pallas_call(
        _wxh_body,
        grid=(n // nb,),
        in_specs=[
            pl.BlockSpec((nb, d), lambda g: (g, 0)),
            pl.BlockSpec((d, d), lambda g: (0, 0)),
            pl.BlockSpec((1, d), lambda g: (0, 0)),
        ],
        out_specs=pl.BlockSpec((nb, d), lambda g: (g, 0)),
        out_shape=jax.ShapeDtypeStruct((n, d), jnp.float32),
    )(x, Wl, bl.reshape(1, d))

    eb = 4000
    f = pl.pallas_call(
        _filter_body,
        grid=(e // eb,),
        in_specs=[
            pl.BlockSpec((eb, k_rbf), lambda g: (g, 0)),
            pl.BlockSpec((k_rbf, d), lambda g: (0, 0)),
            pl.BlockSpec((1, d), lambda g: (0, 0)),
            pl.BlockSpec((d, d), lambda g: (0, 0)),
            pl.BlockSpec((1, d), lambda g: (0, 0)),
        ],
        out_specs=pl.BlockSpec((eb, d), lambda g: (g, 0)),
        out_shape=jax.ShapeDtypeStruct((e, d), jnp.float32),
    )(rbf, W1, b1.reshape(1, d), W2, b2.reshape(1, d))

    mesh = plsc.VectorSubcoreMesh(core_axis_name="c", subcore_axis_name="s")
    agg2 = pl.kernel(
        _sc_body,
        out_type=jax.ShapeDtypeStruct((NC, NDRAIN, d), jnp.float32),
        mesh=mesh,
        compiler_params=pltpu.CompilerParams(needs_layout_passes=False),
        scratch_types=[
            pltpu.VMEM((EPT,), jnp.int32),
            pltpu.VMEM((EPT,), jnp.int32),
            pltpu.VMEM((4096,), jnp.int32),
            pltpu.VMEM((2, CW), jnp.int32),
            pltpu.VMEM((2, CW), jnp.int32),
            pltpu.VMEM((3, CW), jnp.int32),
            pltpu.VMEM((2, CW, d), jnp.float32),
            pltpu.VMEM((2, CW, d), jnp.float32),
            pltpu.VMEM((QR, d), jnp.float32),
            pltpu.VMEM_SHARED((NACC, d), jnp.float32),
            pltpu.SemaphoreType.DMA,
            pltpu.SemaphoreType.DMA,
            pltpu.SemaphoreType.DMA,
        ],
    )(f, wxh, i2, j2)

    nlb = 1000
    out = pl.pallas_call(
        _ln_body,
        grid=(n // nlb,),
        in_specs=[
            pl.BlockSpec((nlb, d), lambda g: (g, 0)),
            pl.BlockSpec((1, nlb, d), lambda g: (g // 5, g % 5, 0)),
            pl.BlockSpec((1, d), lambda g: (0, 0)),
            pl.BlockSpec((1, d), lambda g: (0, 0)),
        ],
        out_specs=pl.BlockSpec((nlb, d), lambda g: (g, 0)),
        out_shape=jax.ShapeDtypeStruct((n, d), jnp.float32),
    )(x, agg2, gamma.reshape(1, d), beta.reshape(1, d))
    return out


# speedup vs baseline: 1.3201x; 1.0488x over previous
import functools

import jax
import jax.numpy as jnp
from jax import lax
from jax.experimental import pallas as pl
from jax.experimental.pallas import tpu as pltpu, tpu_sc as plsc

NC = 2
NS = 16
CW = 80
NSPLIT = 5000
NACC = 5128
NDRAIN = 5120
QR = 16
EPT = 20000
SEG = 4000
NSEG = EPT // SEG
TRASH = 4090


def _wxh_body(x_ref, w_ref, b_ref, o_ref):
    o_ref[...] = (
        jnp.dot(x_ref[...], w_ref[...], preferred_element_type=jnp.float32)
        + b_ref[...]
    )


def _filter_body(rbf_ref, w1_ref, b1_ref, w2_ref, b2_ref, o_ref):
    h = (
        jnp.dot(rbf_ref[...], w1_ref[...], preferred_element_type=jnp.float32)
        + b1_ref[...]
    )
    h = h * jax.nn.sigmoid(h)
    o_ref[...] = (
        jnp.dot(h, w2_ref[...], preferred_element_type=jnp.float32) + b2_ref[...]
    )


def _ln_body(x_ref, a_ref, g_ref, b_ref, o_ref):
    y = x_ref[...] + a_ref[0]
    mu = jnp.mean(y, axis=1, keepdims=True)
    yc = y - mu
    var = jnp.mean(yc * yc, axis=1, keepdims=True)
    o_ref[...] = yc * lax.rsqrt(var + 1e-5) * g_ref[...] + b_ref[...]


def _sc_body(f_h, wxh_h, i_h, j_h, out_h,
             iraw, jidx, comp, fidx2, jrow2, irow3, f_v, rows_v, buf_v, acc,
             gsem, fsem, ssem):
    ci = lax.axis_index("c")
    si = lax.axis_index("s")
    ept = EPT
    drain_per_tile = NDRAIN // NS
    n_q = drain_per_tile // QR
    base = ci * NSPLIT

    def zrow(r, c):
        for cb in range(8):
            buf_v[r, pl.ds(cb * 16, 16)] = jnp.zeros((16,), jnp.float32)
        return c

    lax.fori_loop(0, QR, zrow, 0)
    for q in range(n_q):
        pltpu.sync_copy(buf_v, acc.at[pl.ds(si * drain_per_tile + q * QR, QR)])
    plsc.subcore_barrier()

    pltpu.sync_copy(i_h.at[si], iraw)
    pltpu.sync_copy(j_h.at[si], jidx)

    def build_idx(k, cnt, q, m3):
        for cb in range(CW // 16):
            sl = pl.ds(cb * 16, 16)
            ids = comp[pl.ds(k * CW + cb * 16, 16)]
            fidx2[q, sl] = ids + si * ept
            jrow2[q, sl] = plsc.load_gather(jidx, [ids])
            iv = plsc.load_gather(iraw, [ids])
            pos = k * CW + cb * 16 + lax.iota(jnp.int32, 16)
            irow3[m3, sl] = jnp.where(pos < cnt, iv - base, NSPLIT)

    def issue_f(q, fv):
        return pltpu.async_copy(f_h.at[fidx2.at[q]], fv, fsem)

    def issue_wxh(q, rv):
        return pltpu.async_copy(wxh_h.at[jrow2.at[q]], rv, gsem)

    def wait(sem, dst):
        pltpu.make_async_copy(wxh_h.at[pl.ds(0, dst.shape[0])], dst, sem).wait()

    iota16 = lax.iota(jnp.int32, 16)

    for sg in range(NSEG):
        def czero(g, c):
            comp[pl.ds(g * 16, 16)] = jnp.zeros((16,), jnp.int32)
            return c

        lax.fori_loop(0, (SEG + CW) // 16, czero, 0)

        def cgroup(g, off):
            v = iraw[pl.ds(sg * SEG + g * 16, 16)]
            m = (v >= base) & (v < base + NSPLIT)
            ids = sg * SEG + g * 16 + iota16
            pfx = plsc.cumsum(m.astype(jnp.int32))
            pos = jnp.where(m, off + pfx - 1, TRASH)
            plsc.store_scatter(comp, [pos], ids)
            return off + pfx[15]

        cnt = lax.fori_loop(0, SEG // 16, cgroup, 0)
        nck = (cnt + CW - 1) // CW

        @pl.when(nck > 0)
        def _():
            build_idx(0, cnt, 0, 0)
            issue_f(0, f_v.at[0])
            issue_wxh(0, rows_v.at[0])

        def halfstep(k, p, q):
            @pl.when(k >= 1)
            def _():
                pltpu.make_async_copy(rows_v.at[q], acc.at[pl.ds(0, CW)],
                                      ssem).wait()

            @pl.when(k + 1 < nck)
            def _():
                build_idx(k + 1, cnt, q, lax.rem(k + 1, 3))
                issue_f(q, f_v.at[q])
                issue_wxh(q, rows_v.at[q])

            wait(fsem, f_v.at[p])
            wait(gsem, rows_v.at[p])

            def mrow(r, c2):
                for cb in range(8):
                    sl = pl.ds(cb * 16, 16)
                    rows_v[p, r, sl] = rows_v[p, r, sl] * f_v[p, r, sl]
                return c2

            lax.fori_loop(0, CW, mrow, 0)
            pltpu.async_copy(rows_v.at[p], acc.at[irow3.at[lax.rem(k, 3)]],
                             ssem, add=True)

        def pair(g, c):
            halfstep(2 * g, 0, 1)

            @pl.when(2 * g + 1 < nck)
            def _():
                halfstep(2 * g + 1, 1, 0)

            return c

        lax.fori_loop(0, (nck + 1) // 2, pair, 0)

        @pl.when(nck > 0)
        def _():
            pltpu.make_async_copy(rows_v.at[lax.rem(nck - 1, 2)],
                                  acc.at[pl.ds(0, CW)], ssem).wait()

    plsc.subcore_barrier()

    for q in range(n_q):
        sl = pl.ds(si * drain_per_tile + q * QR, QR)
        pltpu.sync_copy(acc.at[sl], buf_v)
        pltpu.sync_copy(buf_v, out_h.at[ci, sl])


def kernel(x, i, j, rbf, W1, b1, W2, b2, Wl, bl, gamma, beta):
    n, d = x.shape
    e = i.shape[0]
    k_rbf = rbf.shape[1]
    n_chunks = e // (NS * CW)

    i2 = i.astype(jnp.int32).reshape(NS, EPT)
    j2 = j.astype(jnp.int32).reshape(NS, EPT)

    nb = 2000
    wxh = pl.pallas_call(
        _wxh_body,
        grid=(n // nb,),
        in_specs=[
            pl.BlockSpec((nb, d), lambda g: (g, 0)),
            pl.BlockSpec((d, d), lambda g: (0, 0)),
            pl.BlockSpec((1, d), lambda g: (0, 0)),
        ],
        out_specs=pl.BlockSpec((nb, d), lambda g: (g, 0)),
        out_shape=jax.ShapeDtypeStruct((n, d), jnp.float32),
    )(x, Wl, bl.reshape(1, d))

    eb = 8000
    f = pl.pallas_call(
        _filter_body,
        grid=(e // eb,),
        in_specs=[
            pl.BlockSpec((eb, k_rbf), lambda g: (g, 0)),
            pl.BlockSpec((k_rbf, d), lambda g: (0, 0)),
            pl.BlockSpec((1, d), lambda g: (0, 0)),
            pl.BlockSpec((d, d), lambda g: (0, 0)),
            pl.BlockSpec((1, d), lambda g: (0, 0)),
        ],
        out_specs=pl.BlockSpec((eb, d), lambda g: (g, 0)),
        out_shape=jax.ShapeDtypeStruct((e, d), jnp.float32),
    )(rbf, W1, b1.reshape(1, d), W2, b2.reshape(1, d))

    mesh = plsc.VectorSubcoreMesh(core_axis_name="c", subcore_axis_name="s")
    agg2 = pl.kernel(
        _sc_body,
        out_type=jax.ShapeDtypeStruct((NC, NDRAIN, d), jnp.float32),
        mesh=mesh,
        compiler_params=pltpu.CompilerParams(needs_layout_passes=False),
        scratch_types=[
            pltpu.VMEM((EPT,), jnp.int32),
            pltpu.VMEM((EPT,), jnp.int32),
            pltpu.VMEM((4096,), jnp.int32),
            pltpu.VMEM((2, CW), jnp.int32),
            pltpu.VMEM((2, CW), jnp.int32),
            pltpu.VMEM((3, CW), jnp.int32),
            pltpu.VMEM((2, CW, d), jnp.float32),
            pltpu.VMEM((2, CW, d), jnp.float32),
            pltpu.VMEM((QR, d), jnp.float32),
            pltpu.VMEM_SHARED((NACC, d), jnp.float32),
            pltpu.SemaphoreType.DMA,
            pltpu.SemaphoreType.DMA,
            pltpu.SemaphoreType.DMA,
        ],
    )(f, wxh, i2, j2)

    nlb = 1000
    out = pl.pallas_call(
        _ln_body,
        grid=(n // nlb,),
        in_specs=[
            pl.BlockSpec((nlb, d), lambda g: (g, 0)),
            pl.BlockSpec((1, nlb, d), lambda g: (g // 5, g % 5, 0)),
            pl.BlockSpec((1, d), lambda g: (0, 0)),
            pl.BlockSpec((1, d), lambda g: (0, 0)),
        ],
        out_specs=pl.BlockSpec((nlb, d), lambda g: (g, 0)),
        out_shape=jax.ShapeDtypeStruct((n, d), jnp.float32),
    )(x, agg2, gamma.reshape(1, d), beta.reshape(1, d))
    return out


# speedup vs baseline: 1.3229x; 1.0021x over previous
import functools

import jax
import jax.numpy as jnp
from jax import lax
from jax.experimental import pallas as pl
from jax.experimental.pallas import tpu as pltpu, tpu_sc as plsc

NC = 2
NS = 16
CW = 80
NSPLIT = 5000
NACC = 5128
NDRAIN = 5120
QR = 16
EPT = 20000
SEG = 4000
NSEG = EPT // SEG
TRASH = 4090


def _wxh_body(x_ref, w_ref, b_ref, o_ref):
    o_ref[...] = (
        jnp.dot(x_ref[...], w_ref[...], preferred_element_type=jnp.float32)
        + b_ref[...]
    )


def _filter_body(rbf_ref, w1_ref, b1_ref, w2_ref, b2_ref, o_ref):
    h = (
        jnp.dot(rbf_ref[...], w1_ref[...], preferred_element_type=jnp.float32)
        + b1_ref[...]
    )
    h = h * jax.nn.sigmoid(h)
    o_ref[...] = (
        jnp.dot(h, w2_ref[...], preferred_element_type=jnp.float32) + b2_ref[...]
    )


def _ln_body(x_ref, a_ref, g_ref, b_ref, o_ref):
    y = x_ref[...] + a_ref[0]
    mu = jnp.mean(y, axis=1, keepdims=True)
    yc = y - mu
    var = jnp.mean(yc * yc, axis=1, keepdims=True)
    o_ref[...] = yc * lax.rsqrt(var + 1e-5) * g_ref[...] + b_ref[...]


def _sc_body(f_h, wxh_h, i_h, j_h, out_h,
             iraw, jidx, comp, fidx2, jrow2, irow3, f_v, rows_v, buf_v, acc,
             gsem, fsem, ssem):
    ci = lax.axis_index("c")
    si = lax.axis_index("s")
    ept = EPT
    drain_per_tile = NDRAIN // NS
    n_q = drain_per_tile // QR
    base = ci * NSPLIT

    def zrow(r, c):
        for cb in range(8):
            buf_v[r, pl.ds(cb * 16, 16)] = jnp.zeros((16,), jnp.float32)
        return c

    lax.fori_loop(0, QR, zrow, 0)
    for q in range(n_q):
        pltpu.sync_copy(buf_v, acc.at[pl.ds(si * drain_per_tile + q * QR, QR)])
    plsc.subcore_barrier()

    pltpu.sync_copy(i_h.at[si], iraw)
    pltpu.sync_copy(j_h.at[si], jidx)

    def build_idx(k, cnt, q, m3):
        for cb in range(CW // 16):
            sl = pl.ds(cb * 16, 16)
            ids = comp[pl.ds(k * CW + cb * 16, 16)]
            fidx2[q, sl] = ids + si * ept
            jrow2[q, sl] = plsc.load_gather(jidx, [ids])
            iv = plsc.load_gather(iraw, [ids])
            pos = k * CW + cb * 16 + lax.iota(jnp.int32, 16)
            irow3[m3, sl] = jnp.where(pos < cnt, iv - base, NSPLIT)

    def issue_f(q, fv):
        return pltpu.async_copy(f_h.at[fidx2.at[q]], fv, fsem)

    def issue_wxh(q, rv):
        return pltpu.async_copy(wxh_h.at[jrow2.at[q]], rv, gsem)

    def wait(sem, dst):
        pltpu.make_async_copy(wxh_h.at[pl.ds(0, dst.shape[0])], dst, sem).wait()

    iota16 = lax.iota(jnp.int32, 16)

    for sg in range(NSEG):
        def czero(g, c):
            comp[pl.ds(g * 16, 16)] = jnp.zeros((16,), jnp.int32)
            return c

        lax.fori_loop(0, (SEG + CW) // 16, czero, 0)

        def cgroup(g, off):
            v = iraw[pl.ds(sg * SEG + g * 16, 16)]
            m = (v >= base) & (v < base + NSPLIT)
            ids = sg * SEG + g * 16 + iota16
            pfx = plsc.cumsum(m.astype(jnp.int32))
            pos = jnp.where(m, off + pfx - 1, TRASH)
            plsc.store_scatter(comp, [pos], ids)
            return off + pfx[15]

        cnt = lax.fori_loop(0, SEG // 16, cgroup, 0)
        nck = (cnt + CW - 1) // CW

        @pl.when(nck > 0)
        def _():
            build_idx(0, cnt, 0, 0)
            issue_f(0, f_v.at[0])
            issue_wxh(0, rows_v.at[0])

        def halfstep(k, p, q):
            @pl.when(k >= 1)
            def _():
                pltpu.make_async_copy(rows_v.at[q], acc.at[pl.ds(0, CW)],
                                      ssem).wait()

            @pl.when(k + 1 < nck)
            def _():
                build_idx(k + 1, cnt, q, lax.rem(k + 1, 3))
                issue_f(q, f_v.at[q])
                issue_wxh(q, rows_v.at[q])

            wait(fsem, f_v.at[p])
            wait(gsem, rows_v.at[p])

            def mrow(r, c2):
                for cb in range(8):
                    sl = pl.ds(cb * 16, 16)
                    rows_v[p, r, sl] = rows_v[p, r, sl] * f_v[p, r, sl]
                return c2

            lax.fori_loop(0, CW, mrow, 0)
            pltpu.async_copy(rows_v.at[p], acc.at[irow3.at[lax.rem(k, 3)]],
                             ssem, add=True)

        def pair(g, c):
            halfstep(2 * g, 0, 1)

            @pl.when(2 * g + 1 < nck)
            def _():
                halfstep(2 * g + 1, 1, 0)

            return c

        lax.fori_loop(0, (nck + 1) // 2, pair, 0)

        @pl.when(nck > 0)
        def _():
            pltpu.make_async_copy(rows_v.at[lax.rem(nck - 1, 2)],
                                  acc.at[pl.ds(0, CW)], ssem).wait()

    plsc.subcore_barrier()

    for q in range(n_q):
        sl = pl.ds(si * drain_per_tile + q * QR, QR)
        pltpu.sync_copy(acc.at[sl], buf_v)
        pltpu.sync_copy(buf_v, out_h.at[ci, sl])


def kernel(x, i, j, rbf, W1, b1, W2, b2, Wl, bl, gamma, beta):
    n, d = x.shape
    e = i.shape[0]
    k_rbf = rbf.shape[1]
    n_chunks = e // (NS * CW)

    i2 = i.astype(jnp.int32).reshape(NS, EPT)
    j2 = j.astype(jnp.int32).reshape(NS, EPT)

    nb = 2000
    wxh = pl.pallas_call(
        _wxh_body,
        grid=(n // nb,),
        in_specs=[
            pl.BlockSpec((nb, d), lambda g: (g, 0)),
            pl.BlockSpec((d, d), lambda g: (0, 0)),
            pl.BlockSpec((1, d), lambda g: (0, 0)),
        ],
        out_specs=pl.BlockSpec((nb, d), lambda g: (g, 0)),
        out_shape=jax.ShapeDtypeStruct((n, d), jnp.float32),
    )(x, Wl, bl.reshape(1, d))

    eb = 16000
    f = pl.pallas_call(
        _filter_body,
        grid=(e // eb,),
        in_specs=[
            pl.BlockSpec((eb, k_rbf), lambda g: (g, 0)),
            pl.BlockSpec((k_rbf, d), lambda g: (0, 0)),
            pl.BlockSpec((1, d), lambda g: (0, 0)),
            pl.BlockSpec((d, d), lambda g: (0, 0)),
            pl.BlockSpec((1, d), lambda g: (0, 0)),
        ],
        out_specs=pl.BlockSpec((eb, d), lambda g: (g, 0)),
        out_shape=jax.ShapeDtypeStruct((e, d), jnp.float32),
    )(rbf, W1, b1.reshape(1, d), W2, b2.reshape(1, d))

    mesh = plsc.VectorSubcoreMesh(core_axis_name="c", subcore_axis_name="s")
    agg2 = pl.kernel(
        _sc_body,
        out_type=jax.ShapeDtypeStruct((NC, NDRAIN, d), jnp.float32),
        mesh=mesh,
        compiler_params=pltpu.CompilerParams(needs_layout_passes=False),
        scratch_types=[
            pltpu.VMEM((EPT,), jnp.int32),
            pltpu.VMEM((EPT,), jnp.int32),
            pltpu.VMEM((4096,), jnp.int32),
            pltpu.VMEM((2, CW), jnp.int32),
            pltpu.VMEM((2, CW), jnp.int32),
            pltpu.VMEM((3, CW), jnp.int32),
            pltpu.VMEM((2, CW, d), jnp.float32),
            pltpu.VMEM((2, CW, d), jnp.float32),
            pltpu.VMEM((QR, d), jnp.float32),
            pltpu.VMEM_SHARED((NACC, d), jnp.float32),
            pltpu.SemaphoreType.DMA,
            pltpu.SemaphoreType.DMA,
            pltpu.SemaphoreType.DMA,
        ],
    )(f, wxh, i2, j2)

    nlb = 1000
    out = pl.pallas_call(
        _ln_body,
        grid=(n // nlb,),
        in_specs=[
            pl.BlockSpec((nlb, d), lambda g: (g, 0)),
            pl.BlockSpec((1, nlb, d), lambda g: (g // 5, g % 5, 0)),
            pl.BlockSpec((1, d), lambda g: (0, 0)),
            pl.BlockSpec((1, d), lambda g: (0, 0)),
        ],
        out_specs=pl.BlockSpec((nlb, d), lambda g: (g, 0)),
        out_shape=jax.ShapeDtypeStruct((n, d), jnp.float32),
    )(x, agg2, gamma.reshape(1, d), beta.reshape(1, d))
    return out


# speedup vs baseline: 1.3382x; 1.0116x over previous
import functools

import jax
import jax.numpy as jnp
from jax import lax
from jax.experimental import pallas as pl
from jax.experimental.pallas import tpu as pltpu, tpu_sc as plsc

NC = 2
NS = 16
CW = 80
NSPLIT = 5000
NACC = 5128
NDRAIN = 5120
QR = 16
EPT = 20000
SEG = 4000
NSEG = EPT // SEG
TRASH = 4090


def _wxh_body(x_ref, w_ref, b_ref, o_ref):
    o_ref[...] = (
        jnp.dot(x_ref[...], w_ref[...], preferred_element_type=jnp.float32)
        + b_ref[...]
    )


def _filter_body(rbf_ref, w1_ref, b1_ref, w2_ref, b2_ref, o_ref):
    h = (
        jnp.dot(rbf_ref[...], w1_ref[...], preferred_element_type=jnp.float32)
        + b1_ref[...]
    )
    h = h * jax.nn.sigmoid(h)
    o_ref[...] = (
        jnp.dot(h, w2_ref[...], preferred_element_type=jnp.float32) + b2_ref[...]
    )


def _ln_body(x_ref, a_ref, g_ref, b_ref, o_ref):
    y = x_ref[...] + a_ref[0]
    mu = jnp.mean(y, axis=1, keepdims=True)
    yc = y - mu
    var = jnp.mean(yc * yc, axis=1, keepdims=True)
    o_ref[...] = yc * lax.rsqrt(var + 1e-5) * g_ref[...] + b_ref[...]


def _sc_body(f_h, wxh_h, i_h, j_h, out_h,
             iraw, jidx, comp, fidx2, jrow2, irow3, f_v, rows_v, buf_v, acc,
             gsem, fsem, ssem):
    ci = lax.axis_index("c")
    si = lax.axis_index("s")
    ept = EPT
    drain_per_tile = NDRAIN // NS
    n_q = drain_per_tile // QR
    base = ci * NSPLIT

    def zrow(r, c):
        for cb in range(8):
            buf_v[r, pl.ds(cb * 16, 16)] = jnp.zeros((16,), jnp.float32)
        return c

    lax.fori_loop(0, QR, zrow, 0)
    for q in range(n_q):
        pltpu.sync_copy(buf_v, acc.at[pl.ds(si * drain_per_tile + q * QR, QR)])
    plsc.subcore_barrier()

    pltpu.sync_copy(i_h.at[si], iraw)
    pltpu.sync_copy(j_h.at[si], jidx)

    def build_idx(k, cnt, q, m3):
        for cb in range(CW // 16):
            sl = pl.ds(cb * 16, 16)
            ids = comp[pl.ds(k * CW + cb * 16, 16)]
            fidx2[q, sl] = ids + si * ept
            jrow2[q, sl] = plsc.load_gather(jidx, [ids])
            iv = plsc.load_gather(iraw, [ids])
            pos = k * CW + cb * 16 + lax.iota(jnp.int32, 16)
            irow3[m3, sl] = jnp.where(pos < cnt, iv - base, NSPLIT)

    def issue_f(q, fv):
        return pltpu.async_copy(f_h.at[fidx2.at[q]], fv, fsem)

    def issue_wxh(q, rv):
        return pltpu.async_copy(wxh_h.at[jrow2.at[q]], rv, gsem)

    def wait(sem, dst):
        pltpu.make_async_copy(wxh_h.at[pl.ds(0, dst.shape[0])], dst, sem).wait()

    iota16 = lax.iota(jnp.int32, 16)

    for sg in range(NSEG):
        def czero(g, c):
            comp[pl.ds(g * 16, 16)] = jnp.zeros((16,), jnp.int32)
            return c

        lax.fori_loop(0, (SEG + CW) // 16, czero, 0)

        def cgroup(g, off):
            v = iraw[pl.ds(sg * SEG + g * 16, 16)]
            m = (v >= base) & (v < base + NSPLIT)
            ids = sg * SEG + g * 16 + iota16
            pfx = plsc.cumsum(m.astype(jnp.int32))
            pos = jnp.where(m, off + pfx - 1, TRASH)
            plsc.store_scatter(comp, [pos], ids)
            return off + pfx[15]

        cnt = lax.fori_loop(0, SEG // 16, cgroup, 0)
        nck = (cnt + CW - 1) // CW

        @pl.when(nck > 0)
        def _():
            build_idx(0, cnt, 0, 0)
            issue_f(0, f_v.at[0])
            issue_wxh(0, rows_v.at[0])

        def halfstep(k, p, q):
            @pl.when(k + 1 < nck)
            def _():
                build_idx(k + 1, cnt, q, lax.rem(k + 1, 3))

            @pl.when(k >= 1)
            def _():
                pltpu.make_async_copy(rows_v.at[q], acc.at[pl.ds(0, CW)],
                                      ssem).wait()

            @pl.when(k + 1 < nck)
            def _():
                issue_f(q, f_v.at[q])
                issue_wxh(q, rows_v.at[q])

            wait(fsem, f_v.at[p])
            wait(gsem, rows_v.at[p])

            def mrow(r, c2):
                for cb in range(8):
                    sl = pl.ds(cb * 16, 16)
                    rows_v[p, r, sl] = rows_v[p, r, sl] * f_v[p, r, sl]
                return c2

            lax.fori_loop(0, CW, mrow, 0)
            pltpu.async_copy(rows_v.at[p], acc.at[irow3.at[lax.rem(k, 3)]],
                             ssem, add=True)

        def pair(g, c):
            halfstep(2 * g, 0, 1)

            @pl.when(2 * g + 1 < nck)
            def _():
                halfstep(2 * g + 1, 1, 0)

            return c

        lax.fori_loop(0, (nck + 1) // 2, pair, 0)

        @pl.when(nck > 0)
        def _():
            pltpu.make_async_copy(rows_v.at[lax.rem(nck - 1, 2)],
                                  acc.at[pl.ds(0, CW)], ssem).wait()

    plsc.subcore_barrier()

    for q in range(n_q):
        sl = pl.ds(si * drain_per_tile + q * QR, QR)
        pltpu.sync_copy(acc.at[sl], buf_v)
        pltpu.sync_copy(buf_v, out_h.at[ci, sl])


def kernel(x, i, j, rbf, W1, b1, W2, b2, Wl, bl, gamma, beta):
    n, d = x.shape
    e = i.shape[0]
    k_rbf = rbf.shape[1]
    n_chunks = e // (NS * CW)

    i2 = i.astype(jnp.int32).reshape(NS, EPT)
    j2 = j.astype(jnp.int32).reshape(NS, EPT)

    nb = 2000
    wxh = pl.pallas_call(
        _wxh_body,
        grid=(n // nb,),
        in_specs=[
            pl.BlockSpec((nb, d), lambda g: (g, 0)),
            pl.BlockSpec((d, d), lambda g: (0, 0)),
            pl.BlockSpec((1, d), lambda g: (0, 0)),
        ],
        out_specs=pl.BlockSpec((nb, d), lambda g: (g, 0)),
        out_shape=jax.ShapeDtypeStruct((n, d), jnp.float32),
    )(x, Wl, bl.reshape(1, d))

    eb = 16000
    f = pl.pallas_call(
        _filter_body,
        grid=(e // eb,),
        in_specs=[
            pl.BlockSpec((eb, k_rbf), lambda g: (g, 0)),
            pl.BlockSpec((k_rbf, d), lambda g: (0, 0)),
            pl.BlockSpec((1, d), lambda g: (0, 0)),
            pl.BlockSpec((d, d), lambda g: (0, 0)),
            pl.BlockSpec((1, d), lambda g: (0, 0)),
        ],
        out_specs=pl.BlockSpec((eb, d), lambda g: (g, 0)),
        out_shape=jax.ShapeDtypeStruct((e, d), jnp.float32),
    )(rbf, W1, b1.reshape(1, d), W2, b2.reshape(1, d))

    mesh = plsc.VectorSubcoreMesh(core_axis_name="c", subcore_axis_name="s")
    agg2 = pl.kernel(
        _sc_body,
        out_type=jax.ShapeDtypeStruct((NC, NDRAIN, d), jnp.float32),
        mesh=mesh,
        compiler_params=pltpu.CompilerParams(needs_layout_passes=False),
        scratch_types=[
            pltpu.VMEM((EPT,), jnp.int32),
            pltpu.VMEM((EPT,), jnp.int32),
            pltpu.VMEM((4096,), jnp.int32),
            pltpu.VMEM((2, CW), jnp.int32),
            pltpu.VMEM((2, CW), jnp.int32),
            pltpu.VMEM((3, CW), jnp.int32),
            pltpu.VMEM((2, CW, d), jnp.float32),
            pltpu.VMEM((2, CW, d), jnp.float32),
            pltpu.VMEM((QR, d), jnp.float32),
            pltpu.VMEM_SHARED((NACC, d), jnp.float32),
            pltpu.SemaphoreType.DMA,
            pltpu.SemaphoreType.DMA,
            pltpu.SemaphoreType.DMA,
        ],
    )(f, wxh, i2, j2)

    nlb = 1000
    out = pl.pallas_call(
        _ln_body,
        grid=(n // nlb,),
        in_specs=[
            pl.BlockSpec((nlb, d), lambda g: (g, 0)),
            pl.BlockSpec((1, nlb, d), lambda g: (g // 5, g % 5, 0)),
            pl.BlockSpec((1, d), lambda g: (0, 0)),
            pl.BlockSpec((1, d), lambda g: (0, 0)),
        ],
        out_specs=pl.BlockSpec((nlb, d), lambda g: (g, 0)),
        out_shape=jax.ShapeDtypeStruct((n, d), jnp.float32),
    )(x, agg2, gamma.reshape(1, d), beta.reshape(1, d))
    return out
